# baseline (speedup 1.0000x reference)
import functools

import jax
import jax.numpy as jnp
from jax import lax
from jax.experimental import pallas as pl
from jax.experimental.pallas import tpu as pltpu
from jax.experimental.pallas import tpu_sc as plsc

_NC = 2
_NS = 16
_NW = _NC * _NS
_LANES = 16
_CH = 64
_GROUPS = _CH // _LANES
_OFF = 512


def _sc_project(k_rows, n_ray, n_x, n_y, n_z):
    nseg = k_rows - 1
    nrows = nseg // 2
    rays_per_w = n_ray // _NW
    chunks = rays_per_w // _CH
    tv_blk = k_rows * _CH
    par_blk = 7 * _CH

    mesh = plsc.VectorSubcoreMesh(core_axis_name="c", subcore_axis_name="s")

    @functools.partial(
        pl.kernel,
        out_type=jax.ShapeDtypeStruct((n_ray,), jnp.float32),
        mesh=mesh,
        scratch_types=[
            pltpu.VMEM((k_rows * _CH,), jnp.float32),
            pltpu.VMEM((2, nrows, 2 * _CH), jnp.int32),
            pltpu.VMEM((2, nrows, 2 * _CH), jnp.float32),
            pltpu.VMEM((2, nrows, 2 * _CH), jnp.float32),
            pltpu.VMEM((7 * _CH,), jnp.float32),
            pltpu.VMEM((_CH,), jnp.float32),
            pltpu.VMEM((_LANES,), jnp.int32),
            pltpu.SemaphoreType.DMA,
            pltpu.SemaphoreType.DMA,
        ],
    )
    def body(vol_hbm, tvT_hbm, par_hbm, rb_hbm, out_hbm,
             tv_v, idx_v, w_v, val_v, par_v, sino_v, rb_v, sem0, sem1):
        wid = lax.axis_index("s") * _NC + lax.axis_index("c")
        base = wid * rays_per_w
        blk0 = wid * chunks
        pltpu.sync_copy(rb_hbm.at[pl.ds(wid * _LANES, _LANES)], rb_v)
        rb_vec = rb_v[pl.ds(0, _LANES)]
        sems = (sem0, sem1)
        one = jnp.float32(1.0)
        oob = jnp.int32(~(n_x - 1))
        nv_mask = jnp.int32(n_x * n_y * n_z - 1)
        jitb = wid * 65537 + lax.iota(jnp.int32, _LANES) * 33791

        def pass1(c):
            p = c % 2
            blk = blk0 + c
            pltpu.sync_copy(tvT_hbm.at[pl.ds(blk * tv_blk, tv_blk)], tv_v)
            pltpu.sync_copy(par_hbm.at[pl.ds(blk * par_blk, par_blk)], par_v)
            par = []
            for g in range(_GROUPS):
                par.append(tuple(
                    par_v[pl.ds(i * _CH + g * _LANES, _LANES)]
                    for i in range(7)))

            rbound = rb_vec[c]

            def jbody(r, _):
                for g in range(_GROUPS):
                    gl = g * _LANES
                    rows = [tv_v[pl.ds((2 * r + k) * _CH + gl, _LANES)]
                            for k in range(3)]
                    p0x, p0y, p0z, hx, hy, hz, rl = par[g]
                    for half in range(2):
                        ds = pl.ds(half * _CH + gl, _LANES)
                        t0 = rows[half]
                        t1 = rows[half + 1]
                        s = t0 + t1
                        seg = t1 - t0
                        ixm = (p0x + s * hx).astype(jnp.int32) - _OFF
                        iym = (p0y + s * hy).astype(jnp.int32) - _OFF
                        izm = (p0z + s * hz).astype(jnp.int32) - _OFF
                        ok = ((ixm | iym | izm) & oob) == 0
                        flat = (ixm * n_y + iym) * n_z + izm
                        jit = jitb + (2 * r + half) * 4099
                        safe = (flat + jit) & nv_mask
                        idx_v[p, r, ds] = jnp.where(ok, flat, safe)
                        w_v[p, r, ds] = jnp.where(ok & (t1 <= one),
                                                  seg * rl, 0.0)
                return 0

            lax.fori_loop(0, rbound, jbody, 0)

            def fire(r, _):
                pltpu.make_async_copy(
                    vol_hbm.at[idx_v.at[p, r]], val_v.at[p, r], sems[p]
                ).start()
                return 0

            lax.fori_loop(0, rbound, fire, 0)
            return rbound

        def drain_and_pass2(c, rbase, rbound):
            p = c % 2

            def jbody(r, accs):
                pltpu.make_async_copy(
                    vol_hbm.at[idx_v.at[p, r]], val_v.at[p, r], sems[p]
                ).wait()
                out = list(accs)
                for half in range(2):
                    for g in range(_GROUPS):
                        ds = pl.ds(half * _CH + g * _LANES, _LANES)
                        out[g] = out[g] + val_v[p, r, ds] * w_v[p, r, ds]
                return tuple(out)

            zeros = tuple(jnp.zeros((_LANES,), jnp.float32)
                          for _ in range(_GROUPS))
            accs = lax.fori_loop(0, rbound, jbody, zeros)
            for g in range(_GROUPS):
                sino_v[pl.ds(g * _LANES, _LANES)] = accs[g]
            pltpu.sync_copy(sino_v, out_hbm.at[pl.ds(rbase, _CH)])

        bounds = []
        for c in range(chunks):
            bounds.append(pass1(c))
            if c > 0:
                drain_and_pass2(c - 1, base + (c - 1) * _CH, bounds[c - 1])
        drain_and_pass2(chunks - 1, base + (chunks - 1) * _CH,
                        bounds[chunks - 1])

    return body


def kernel(volume, tvals, M, b, src, dst):
    n_x, n_y, n_z = volume.shape
    n_ray, k_rows = tvals.shape
    src_t = src @ M.T + b.reshape(1, 3)
    dst_t = dst @ M.T + b.reshape(1, 3)
    d = dst_t - src_t
    ray_len = jnp.sqrt(jnp.sum(d * d, axis=1))
    params = jnp.concatenate(
        [src_t.T + jnp.float32(_OFF), 0.5 * d.T, ray_len[None, :]], axis=0)
    n_blk = n_ray // _CH
    tv_blocks = tvals.T.reshape(k_rows, n_blk, _CH).transpose(1, 0, 2).reshape(-1)
    par_blocks = params.reshape(7, n_blk, _CH).transpose(1, 0, 2).reshape(-1)
    vol_flat = volume.reshape(-1)
    cnt = jnp.sum((tvals <= 1.0).astype(jnp.int32), axis=1)
    cmax = jnp.max(cnt.reshape(n_blk, _CH), axis=1)
    rb_blocks = jnp.clip(cmax // 2, 1, (k_rows - 1) // 2)
    chunks_per_w = n_blk // _NW
    rb_blocks = jnp.pad(rb_blocks.reshape(_NW, chunks_per_w),
                        ((0, 0), (0, _LANES - chunks_per_w))).reshape(-1)
    body = _sc_project(k_rows, n_ray, n_x, n_y, n_z)
    return body(vol_flat, tv_blocks, par_blocks, rb_blocks)


# speedup vs baseline: 1.2819x; 1.2819x over previous
import functools

import jax
import jax.numpy as jnp
from jax import lax
from jax.experimental import pallas as pl
from jax.experimental.pallas import tpu as pltpu
from jax.experimental.pallas import tpu_sc as plsc

_NC = 2
_NS = 16
_NW = _NC * _NS
_LANES = 16
_CH = 64
_GROUPS = _CH // _LANES
_OFF = 512


def _sc_project(k_rows, n_ray, n_x, n_y, n_z):
    nseg = k_rows - 1
    nrows = nseg // 2
    rays_per_w = n_ray // _NW
    chunks = rays_per_w // _CH
    tv_blk = k_rows * _CH
    par_blk = 7 * _CH

    mesh = plsc.VectorSubcoreMesh(core_axis_name="c", subcore_axis_name="s")

    @functools.partial(
        pl.kernel,
        out_type=jax.ShapeDtypeStruct((n_ray,), jnp.float32),
        mesh=mesh,
        scratch_types=[
            pltpu.VMEM((k_rows * _CH,), jnp.float32),
            pltpu.VMEM((2, nrows, 2 * _CH), jnp.int32),
            pltpu.VMEM((2, nrows, 2 * _CH), jnp.float32),
            pltpu.VMEM((2, nrows, 2 * _CH), jnp.float32),
            pltpu.VMEM((7 * _CH,), jnp.float32),
            pltpu.VMEM((_CH,), jnp.float32),
            pltpu.VMEM((_LANES,), jnp.int32),
            pltpu.SemaphoreType.DMA,
            pltpu.SemaphoreType.DMA,
        ],
    )
    def body(vol_hbm, tvT_hbm, par_hbm, rb_hbm, out_hbm,
             tv_v, idx_v, w_v, val_v, par_v, sino_v, rb_v, sem0, sem1):
        wid = lax.axis_index("s") * _NC + lax.axis_index("c")
        base = wid * rays_per_w
        blk0 = wid * chunks
        pltpu.sync_copy(rb_hbm.at[pl.ds(wid * _LANES, _LANES)], rb_v)
        rb_vec = rb_v[pl.ds(0, _LANES)]
        sems = (sem0, sem1)
        one = jnp.float32(1.0)
        oob = jnp.int32(~(n_x - 1))
        nv_mask = jnp.int32(n_x * n_y * n_z - 1)
        jitb = wid * 65537 + lax.iota(jnp.int32, _LANES) * 33791

        def pass1(c):
            p = c % 2
            blk = blk0 + c
            pltpu.sync_copy(tvT_hbm.at[pl.ds(blk * tv_blk, tv_blk)], tv_v)
            pltpu.sync_copy(par_hbm.at[pl.ds(blk * par_blk, par_blk)], par_v)
            par = []
            for g in range(_GROUPS):
                par.append(tuple(
                    par_v[pl.ds(i * _CH + g * _LANES, _LANES)]
                    for i in range(7)))

            rbound = rb_vec[c]

            def jbody(r, _):
                for g in range(_GROUPS):
                    gl = g * _LANES
                    rows = [tv_v[pl.ds((2 * r + k) * _CH + gl, _LANES)]
                            for k in range(3)]
                    p0x, p0y, p0z, hx, hy, hz, rl = par[g]
                    for half in range(2):
                        ds = pl.ds(half * _CH + gl, _LANES)
                        t0 = rows[half]
                        t1 = rows[half + 1]
                        s = t0 + t1
                        seg = t1 - t0
                        ixm = (p0x + s * hx).astype(jnp.int32) - _OFF
                        iym = (p0y + s * hy).astype(jnp.int32) - _OFF
                        izm = (p0z + s * hz).astype(jnp.int32) - _OFF
                        ok = ((ixm | iym | izm) & oob) == 0
                        flat = (ixm * n_y + iym) * n_z + izm
                        jit = jitb + (2 * r + half) * 4099
                        safe = (flat + jit) & nv_mask
                        idx_v[p, r, ds] = jnp.where(ok, flat, safe)
                        w_v[p, r, ds] = jnp.where(ok & (t1 <= one),
                                                  seg * rl, 0.0)
                pltpu.make_async_copy(
                    vol_hbm.at[idx_v.at[p, r]], val_v.at[p, r], sems[p]
                ).start()
                return 0

            lax.fori_loop(0, rbound, jbody, 0)
            return rbound

        def drain_and_pass2(c, rbase, rbound):
            p = c % 2

            def jbody(r, accs):
                pltpu.make_async_copy(
                    vol_hbm.at[idx_v.at[p, r]], val_v.at[p, r], sems[p]
                ).wait()
                out = list(accs)
                for half in range(2):
                    for g in range(_GROUPS):
                        ds = pl.ds(half * _CH + g * _LANES, _LANES)
                        out[g] = out[g] + val_v[p, r, ds] * w_v[p, r, ds]
                return tuple(out)

            zeros = tuple(jnp.zeros((_LANES,), jnp.float32)
                          for _ in range(_GROUPS))
            accs = lax.fori_loop(0, rbound, jbody, zeros)
            for g in range(_GROUPS):
                sino_v[pl.ds(g * _LANES, _LANES)] = accs[g]
            pltpu.sync_copy(sino_v, out_hbm.at[pl.ds(rbase, _CH)])

        bounds = []
        for c in range(chunks):
            bounds.append(pass1(c))
            if c > 0:
                drain_and_pass2(c - 1, base + (c - 1) * _CH, bounds[c - 1])
        drain_and_pass2(chunks - 1, base + (chunks - 1) * _CH,
                        bounds[chunks - 1])

    return body


def kernel(volume, tvals, M, b, src, dst):
    n_x, n_y, n_z = volume.shape
    n_ray, k_rows = tvals.shape
    src_t = src @ M.T + b.reshape(1, 3)
    dst_t = dst @ M.T + b.reshape(1, 3)
    d = dst_t - src_t
    ray_len = jnp.sqrt(jnp.sum(d * d, axis=1))
    params = jnp.concatenate(
        [src_t.T + jnp.float32(_OFF), 0.5 * d.T, ray_len[None, :]], axis=0)
    n_blk = n_ray // _CH
    tv_blocks = tvals.T.reshape(k_rows, n_blk, _CH).transpose(1, 0, 2).reshape(-1)
    par_blocks = params.reshape(7, n_blk, _CH).transpose(1, 0, 2).reshape(-1)
    vol_flat = volume.reshape(-1)
    cnt = jnp.sum((tvals <= 1.0).astype(jnp.int32), axis=1)
    cmax = jnp.max(cnt.reshape(n_blk, _CH), axis=1)
    rb_blocks = jnp.clip(cmax // 2, 1, (k_rows - 1) // 2)
    chunks_per_w = n_blk // _NW
    rb_blocks = jnp.pad(rb_blocks.reshape(_NW, chunks_per_w),
                        ((0, 0), (0, _LANES - chunks_per_w))).reshape(-1)
    body = _sc_project(k_rows, n_ray, n_x, n_y, n_z)
    return body(vol_flat, tv_blocks, par_blocks, rb_blocks)
